# baseline (speedup 1.0000x reference)
import functools

import jax
import jax.numpy as jnp
from jax import lax
from jax.experimental import pallas as pl
from jax.experimental.pallas import tpu as pltpu
from jax.experimental.pallas import tpu_sc as plsc

_R = 256
_K = 6
_LANES = 16
_SUBCORES = 32
_N_SC = 4096


def _topk_mask(d2, N):
    it = lax.broadcasted_iota(jnp.int32, d2.shape, 1)
    M = jnp.zeros(d2.shape, jnp.float32)
    for _ in range(_K):
        m = jnp.min(d2, axis=1, keepdims=True)
        idx = jnp.min(jnp.where(d2 == m, it, N), axis=1, keepdims=True)
        sel = it == idx
        M = M + sel.astype(jnp.float32)
        d2 = jnp.where(sel, jnp.inf, d2)
    return M


def _tc_select_body(xyr_ref, xy_ref, x0_ref, S_ref):
    N = xy_ref.shape[1]
    xi = xyr_ref[...]
    xj = xy_ref[0]
    dx = xi[:, 0:1] - xj[:, 0][None, :]
    dy = xi[:, 1:2] - xj[:, 1][None, :]
    M = _topk_mask(dx * dx + dy * dy, N)
    S_ref[...] = jnp.dot(M, x0_ref[...], preferred_element_type=jnp.float32)


def _sc_select_kernel(n_rows):
    rows_per_tile = n_rows // _SUBCORES
    groups = rows_per_tile // _LANES
    mesh = plsc.VectorSubcoreMesh(core_axis_name="c", subcore_axis_name="s")

    def body(qx_hbm, qy_hbm, cx_hbm, cy_hbm, p0_hbm, p1_hbm, p2_hbm,
             s0_hbm, s1_hbm, s2_hbm,
             qx_v, qy_v, cx_v, cy_v, p0_v, p1_v, p2_v, s0_v, s1_v, s2_v):
        N = cx_hbm.shape[0]
        w = lax.axis_index("s") * 2 + lax.axis_index("c")
        base = w * rows_per_tile
        pltpu.sync_copy(qx_hbm.at[pl.ds(base, rows_per_tile)], qx_v)
        pltpu.sync_copy(qy_hbm.at[pl.ds(base, rows_per_tile)], qy_v)
        pltpu.sync_copy(cx_hbm, cx_v)
        pltpu.sync_copy(cy_hbm, cy_v)
        pltpu.sync_copy(p0_hbm, p0_v)
        pltpu.sync_copy(p1_hbm, p1_v)
        pltpu.sync_copy(p2_hbm, p2_v)
        for g in range(groups):
            xi = qx_v[pl.ds(g * _LANES, _LANES)]
            yi = qy_v[pl.ds(g * _LANES, _LANES)]
            inf = jnp.full((_LANES,), jnp.inf, jnp.float32)
            zero = jnp.zeros((_LANES,), jnp.int32)
            carry = (inf, inf, inf, inf, inf, inf,
                     zero, zero, zero, zero, zero, zero)

            def step(c, cr):
                b5 = cr[5]
                jv = jnp.full((_LANES,), c, jnp.int32)
                bx = plsc.load_gather(cx_v, [jv])
                by = plsc.load_gather(cy_v, [jv])
                dx = bx - xi
                dy = by - yi
                d2 = dx * dx + dy * dy
                c5 = d2 < b5

                def insert(_):
                    b0, b1, b2, b3, b4, b5, i0, i1, i2, i3, i4, i5 = cr
                    c0 = d2 < b0
                    c1 = d2 < b1
                    c2 = d2 < b2
                    c3 = d2 < b3
                    c4 = d2 < b4
                    nb5 = jnp.where(c5, jnp.where(c4, b4, d2), b5)
                    ni5 = jnp.where(c5, jnp.where(c4, i4, jv), i5)
                    nb4 = jnp.where(c4, jnp.where(c3, b3, d2), b4)
                    ni4 = jnp.where(c4, jnp.where(c3, i3, jv), i4)
                    nb3 = jnp.where(c3, jnp.where(c2, b2, d2), b3)
                    ni3 = jnp.where(c3, jnp.where(c2, i2, jv), i3)
                    nb2 = jnp.where(c2, jnp.where(c1, b1, d2), b2)
                    ni2 = jnp.where(c2, jnp.where(c1, i1, jv), i2)
                    nb1 = jnp.where(c1, jnp.where(c0, b0, d2), b1)
                    ni1 = jnp.where(c1, jnp.where(c0, i0, jv), i1)
                    nb0 = jnp.where(c0, d2, b0)
                    ni0 = jnp.where(c0, jv, i0)
                    return (nb0, nb1, nb2, nb3, nb4, nb5,
                            ni0, ni1, ni2, ni3, ni4, ni5)

                return lax.cond(jnp.any(c5), insert, lambda _: cr, 0)

            carry = lax.fori_loop(0, N, step, carry, unroll=4)
            idxs = carry[6:]
            s0 = jnp.zeros((_LANES,), jnp.float32)
            s1 = jnp.zeros((_LANES,), jnp.float32)
            s2 = jnp.zeros((_LANES,), jnp.float32)
            for ik in idxs:
                s0 = s0 + plsc.load_gather(p0_v, [ik])
                s1 = s1 + plsc.load_gather(p1_v, [ik])
                s2 = s2 + plsc.load_gather(p2_v, [ik])
            s0_v[pl.ds(g * _LANES, _LANES)] = s0
            s1_v[pl.ds(g * _LANES, _LANES)] = s1
            s2_v[pl.ds(g * _LANES, _LANES)] = s2
        pltpu.sync_copy(s0_v, s0_hbm.at[pl.ds(base, rows_per_tile)])
        pltpu.sync_copy(s1_v, s1_hbm.at[pl.ds(base, rows_per_tile)])
        pltpu.sync_copy(s2_v, s2_hbm.at[pl.ds(base, rows_per_tile)])

    out = jax.ShapeDtypeStruct((n_rows,), jnp.float32)
    vec = lambda n: pltpu.VMEM((n,), jnp.float32)
    return pl.kernel(
        body,
        mesh=mesh,
        compiler_params=pltpu.CompilerParams(needs_layout_passes=False),
        out_type=[out, out, out],
        scratch_types=[
            vec(rows_per_tile), vec(rows_per_tile),
            vec(4096), vec(4096), vec(4096), vec(4096), vec(4096),
            vec(rows_per_tile), vec(rows_per_tile), vec(rows_per_tile),
        ],
    )


def _mlp_body(x_ref, S_ref, depot_ref,
              Wi_ref, bi_ref, Wn_ref, bn_ref, Wf_ref, bf_ref, Wd_ref, bd_ref,
              F_ref, dep_ref, sm_ref):
    j = pl.program_id(1)
    xr = x_ref[0]
    S = S_ref[0]
    F0 = jnp.dot(xr, Wi_ref[...], preferred_element_type=jnp.float32) + bi_ref[...]
    G = F0 + jnp.dot(S - 6.0 * xr, Wn_ref[...],
                     preferred_element_type=jnp.float32) + 6.0 * bn_ref[...]
    F = jnp.dot(G, Wf_ref[...], preferred_element_type=jnp.float32) + 7.0 * bf_ref[...]
    F_ref[0] = F
    rowsum = jnp.sum(F, axis=0, keepdims=True)[None]

    @pl.when(j == 0)
    def _():
        dep_ref[0] = jnp.dot(depot_ref[0], Wd_ref[...],
                             preferred_element_type=jnp.float32) + bd_ref[...]
        sm_ref[...] = rowsum

    @pl.when(j != 0)
    def _():
        sm_ref[...] += rowsum


def kernel(loc, deadline, depot, Wi, bi, Wn, bn, Wf, bf, Wd, bd):
    B, N, _ = loc.shape
    E = Wi.shape[1]
    x = jnp.concatenate([loc, deadline[:, :, None]], axis=2)
    x0 = x[0]
    depot3 = depot[:, None, :]
    bi2, bn2, bf2, bd2 = (v.reshape(1, E) for v in (bi, bn, bf, bd))
    nb = N // _R

    n_tc = B * N - _N_SC
    xq = loc.reshape(B * N, 2)[:n_tc]

    S_tc = pl.pallas_call(
        _tc_select_body,
        grid=(n_tc // _R,),
        in_specs=[
            pl.BlockSpec((_R, 2), lambda j: (j, 0)),
            pl.BlockSpec((1, N, 2), lambda j: (j // (N // _R), 0, 0)),
            pl.BlockSpec((N, 3), lambda j: (0, 0)),
        ],
        out_specs=pl.BlockSpec((_R, 3), lambda j: (j, 0)),
        out_shape=jax.ShapeDtypeStruct((n_tc, 3), jnp.float32),
    )(xq, loc, x0)

    tail = N - _N_SC
    s0, s1, s2 = _sc_select_kernel(_N_SC)(
        loc[1, tail:, 0], loc[1, tail:, 1], loc[1, :, 0], loc[1, :, 1],
        x0[:, 0], x0[:, 1], x0[:, 2])
    S_sc = jnp.stack([s0, s1, s2], axis=-1)
    S = jnp.concatenate([S_tc, S_sc], axis=0).reshape(B, N, 3)

    F, dep, sm = pl.pallas_call(
        _mlp_body,
        grid=(B, nb),
        in_specs=[
            pl.BlockSpec((1, _R, 3), lambda b, j: (b, j, 0)),
            pl.BlockSpec((1, _R, 3), lambda b, j: (b, j, 0)),
            pl.BlockSpec((1, 1, 2), lambda b, j: (b, 0, 0)),
            pl.BlockSpec((3, E), lambda b, j: (0, 0)),
            pl.BlockSpec((1, E), lambda b, j: (0, 0)),
            pl.BlockSpec((3, E), lambda b, j: (0, 0)),
            pl.BlockSpec((1, E), lambda b, j: (0, 0)),
            pl.BlockSpec((E, E), lambda b, j: (0, 0)),
            pl.BlockSpec((1, E), lambda b, j: (0, 0)),
            pl.BlockSpec((2, E), lambda b, j: (0, 0)),
            pl.BlockSpec((1, E), lambda b, j: (0, 0)),
        ],
        out_specs=[
            pl.BlockSpec((1, _R, E), lambda b, j: (b, j, 0)),
            pl.BlockSpec((1, 1, E), lambda b, j: (b, 0, 0)),
            pl.BlockSpec((1, 1, E), lambda b, j: (b, 0, 0)),
        ],
        out_shape=[
            jax.ShapeDtypeStruct((B, N, E), jnp.float32),
            jax.ShapeDtypeStruct((B, 1, E), jnp.float32),
            jax.ShapeDtypeStruct((B, 1, E), jnp.float32),
        ],
    )(x, S, depot3, Wi, bi2, Wn, bn2, Wf, bf2, Wd, bd2)

    h = jnp.concatenate([dep, F], axis=1)
    mean = (dep[:, 0, :] + sm[:, 0, :]) / (N + 1)
    return (h, mean)


# speedup vs baseline: 2.2107x; 2.2107x over previous
import functools

import jax
import jax.numpy as jnp
from jax import lax
from jax.experimental import pallas as pl
from jax.experimental.pallas import tpu as pltpu
from jax.experimental.pallas import tpu_sc as plsc

_R = 256
_K = 6
_LANES = 16
_SUBCORES = 32
_N_SC = 3584


def _topk_mask(d2, N):
    it = lax.broadcasted_iota(jnp.int32, d2.shape, 1)
    M = jnp.zeros(d2.shape, jnp.float32)
    for _ in range(_K):
        m = jnp.min(d2, axis=1, keepdims=True)
        idx = jnp.min(jnp.where(d2 == m, it, N), axis=1, keepdims=True)
        sel = it == idx
        M = M + sel.astype(jnp.float32)
        d2 = jnp.where(sel, jnp.inf, d2)
    return M


def _tc_select_body(xyr_ref, xy_ref, x0_ref, S_ref):
    N = xy_ref.shape[1]
    xi = xyr_ref[...]
    xj = xy_ref[0]
    dx = xi[:, 0:1] - xj[:, 0][None, :]
    dy = xi[:, 1:2] - xj[:, 1][None, :]
    M = _topk_mask(dx * dx + dy * dy, N)
    S_ref[...] = jnp.dot(M, x0_ref[...], preferred_element_type=jnp.float32)


def _sc_select_kernel(n_rows):
    rows_per_tile = n_rows // _SUBCORES
    groups = rows_per_tile // _LANES
    mesh = plsc.VectorSubcoreMesh(core_axis_name="c", subcore_axis_name="s")

    def body(qx_hbm, qy_hbm, cx_hbm, cy_hbm, p0_hbm, p1_hbm, p2_hbm,
             s0_hbm, s1_hbm, s2_hbm,
             qx_v, qy_v, cx_v, cy_v, p0_v, p1_v, p2_v, s0_v, s1_v, s2_v):
        N = cx_hbm.shape[0]
        w = lax.axis_index("s") * 2 + lax.axis_index("c")
        base = w * rows_per_tile
        pltpu.sync_copy(qx_hbm.at[pl.ds(base, rows_per_tile)], qx_v)
        pltpu.sync_copy(qy_hbm.at[pl.ds(base, rows_per_tile)], qy_v)
        pltpu.sync_copy(cx_hbm, cx_v)
        pltpu.sync_copy(cy_hbm, cy_v)
        pltpu.sync_copy(p0_hbm, p0_v)
        pltpu.sync_copy(p1_hbm, p1_v)
        pltpu.sync_copy(p2_hbm, p2_v)
        for g in range(groups):
            xi = qx_v[pl.ds(g * _LANES, _LANES)]
            yi = qy_v[pl.ds(g * _LANES, _LANES)]
            inf = jnp.full((_LANES,), jnp.inf, jnp.float32)
            zero = jnp.zeros((_LANES,), jnp.int32)
            carry = (inf, inf, inf, inf, inf, inf,
                     zero, zero, zero, zero, zero, zero)

            def step(c, cr):
                b0, b1, b2, b3, b4, b5, i0, i1, i2, i3, i4, i5 = cr
                jv = jnp.full((_LANES,), c, jnp.int32)
                bx = plsc.load_gather(cx_v, [jv])
                by = plsc.load_gather(cy_v, [jv])
                dx = bx - xi
                dy = by - yi
                d2 = dx * dx + dy * dy
                c0 = d2 < b0
                c1 = d2 < b1
                c2 = d2 < b2
                c3 = d2 < b3
                c4 = d2 < b4
                c5 = d2 < b5
                nb5 = jnp.where(c5, jnp.where(c4, b4, d2), b5)
                ni5 = jnp.where(c5, jnp.where(c4, i4, jv), i5)
                nb4 = jnp.where(c4, jnp.where(c3, b3, d2), b4)
                ni4 = jnp.where(c4, jnp.where(c3, i3, jv), i4)
                nb3 = jnp.where(c3, jnp.where(c2, b2, d2), b3)
                ni3 = jnp.where(c3, jnp.where(c2, i2, jv), i3)
                nb2 = jnp.where(c2, jnp.where(c1, b1, d2), b2)
                ni2 = jnp.where(c2, jnp.where(c1, i1, jv), i2)
                nb1 = jnp.where(c1, jnp.where(c0, b0, d2), b1)
                ni1 = jnp.where(c1, jnp.where(c0, i0, jv), i1)
                nb0 = jnp.where(c0, d2, b0)
                ni0 = jnp.where(c0, jv, i0)
                return (nb0, nb1, nb2, nb3, nb4, nb5,
                        ni0, ni1, ni2, ni3, ni4, ni5)

            carry = lax.fori_loop(0, N, step, carry, unroll=8)
            idxs = carry[6:]
            s0 = jnp.zeros((_LANES,), jnp.float32)
            s1 = jnp.zeros((_LANES,), jnp.float32)
            s2 = jnp.zeros((_LANES,), jnp.float32)
            for ik in idxs:
                s0 = s0 + plsc.load_gather(p0_v, [ik])
                s1 = s1 + plsc.load_gather(p1_v, [ik])
                s2 = s2 + plsc.load_gather(p2_v, [ik])
            s0_v[pl.ds(g * _LANES, _LANES)] = s0
            s1_v[pl.ds(g * _LANES, _LANES)] = s1
            s2_v[pl.ds(g * _LANES, _LANES)] = s2
        pltpu.sync_copy(s0_v, s0_hbm.at[pl.ds(base, rows_per_tile)])
        pltpu.sync_copy(s1_v, s1_hbm.at[pl.ds(base, rows_per_tile)])
        pltpu.sync_copy(s2_v, s2_hbm.at[pl.ds(base, rows_per_tile)])

    out = jax.ShapeDtypeStruct((n_rows,), jnp.float32)
    vec = lambda n: pltpu.VMEM((n,), jnp.float32)
    return pl.kernel(
        body,
        mesh=mesh,
        compiler_params=pltpu.CompilerParams(needs_layout_passes=False),
        out_type=[out, out, out],
        scratch_types=[
            vec(rows_per_tile), vec(rows_per_tile),
            vec(4096), vec(4096), vec(4096), vec(4096), vec(4096),
            vec(rows_per_tile), vec(rows_per_tile), vec(rows_per_tile),
        ],
    )


def _mlp_body(x_ref, S_ref, depot_ref,
              Wi_ref, bi_ref, Wn_ref, bn_ref, Wf_ref, bf_ref, Wd_ref, bd_ref,
              F_ref, dep_ref, sm_ref):
    j = pl.program_id(1)
    xr = x_ref[0]
    S = S_ref[0]
    F0 = jnp.dot(xr, Wi_ref[...], preferred_element_type=jnp.float32) + bi_ref[...]
    G = F0 + jnp.dot(S - 6.0 * xr, Wn_ref[...],
                     preferred_element_type=jnp.float32) + 6.0 * bn_ref[...]
    F = jnp.dot(G, Wf_ref[...], preferred_element_type=jnp.float32) + 7.0 * bf_ref[...]
    F_ref[0] = F
    rowsum = jnp.sum(F, axis=0, keepdims=True)[None]

    @pl.when(j == 0)
    def _():
        dep_ref[0] = jnp.dot(depot_ref[0], Wd_ref[...],
                             preferred_element_type=jnp.float32) + bd_ref[...]
        sm_ref[...] = rowsum

    @pl.when(j != 0)
    def _():
        sm_ref[...] += rowsum


def kernel(loc, deadline, depot, Wi, bi, Wn, bn, Wf, bf, Wd, bd):
    B, N, _ = loc.shape
    E = Wi.shape[1]
    x = jnp.concatenate([loc, deadline[:, :, None]], axis=2)
    x0 = x[0]
    depot3 = depot[:, None, :]
    bi2, bn2, bf2, bd2 = (v.reshape(1, E) for v in (bi, bn, bf, bd))
    nb = N // _R

    n_tc = B * N - _N_SC
    xq = loc.reshape(B * N, 2)[:n_tc]

    S_tc = pl.pallas_call(
        _tc_select_body,
        grid=(n_tc // _R,),
        in_specs=[
            pl.BlockSpec((_R, 2), lambda j: (j, 0)),
            pl.BlockSpec((1, N, 2), lambda j: (j // (N // _R), 0, 0)),
            pl.BlockSpec((N, 3), lambda j: (0, 0)),
        ],
        out_specs=pl.BlockSpec((_R, 3), lambda j: (j, 0)),
        out_shape=jax.ShapeDtypeStruct((n_tc, 3), jnp.float32),
    )(xq, loc, x0)

    tail = N - _N_SC
    s0, s1, s2 = _sc_select_kernel(_N_SC)(
        loc[1, tail:, 0], loc[1, tail:, 1], loc[1, :, 0], loc[1, :, 1],
        x0[:, 0], x0[:, 1], x0[:, 2])
    S_sc = jnp.stack([s0, s1, s2], axis=-1)
    S = jnp.concatenate([S_tc, S_sc], axis=0).reshape(B, N, 3)

    F, dep, sm = pl.pallas_call(
        _mlp_body,
        grid=(B, nb),
        in_specs=[
            pl.BlockSpec((1, _R, 3), lambda b, j: (b, j, 0)),
            pl.BlockSpec((1, _R, 3), lambda b, j: (b, j, 0)),
            pl.BlockSpec((1, 1, 2), lambda b, j: (b, 0, 0)),
            pl.BlockSpec((3, E), lambda b, j: (0, 0)),
            pl.BlockSpec((1, E), lambda b, j: (0, 0)),
            pl.BlockSpec((3, E), lambda b, j: (0, 0)),
            pl.BlockSpec((1, E), lambda b, j: (0, 0)),
            pl.BlockSpec((E, E), lambda b, j: (0, 0)),
            pl.BlockSpec((1, E), lambda b, j: (0, 0)),
            pl.BlockSpec((2, E), lambda b, j: (0, 0)),
            pl.BlockSpec((1, E), lambda b, j: (0, 0)),
        ],
        out_specs=[
            pl.BlockSpec((1, _R, E), lambda b, j: (b, j, 0)),
            pl.BlockSpec((1, 1, E), lambda b, j: (b, 0, 0)),
            pl.BlockSpec((1, 1, E), lambda b, j: (b, 0, 0)),
        ],
        out_shape=[
            jax.ShapeDtypeStruct((B, N, E), jnp.float32),
            jax.ShapeDtypeStruct((B, 1, E), jnp.float32),
            jax.ShapeDtypeStruct((B, 1, E), jnp.float32),
        ],
    )(x, S, depot3, Wi, bi2, Wn, bn2, Wf, bf2, Wd, bd2)

    h = jnp.concatenate([dep, F], axis=1)
    mean = (dep[:, 0, :] + sm[:, 0, :]) / (N + 1)
    return (h, mean)


# speedup vs baseline: 2.5088x; 1.1349x over previous
import jax
import jax.numpy as jnp
from jax import lax
from jax.experimental import pallas as pl
from jax.experimental.pallas import tpu as pltpu
from jax.experimental.pallas import tpu_sc as plsc

_R = 256
_K = 6
_LANES = 16
_SUBCORES = 32
_N_SC = 3584


def _topk_mask(d2, N):
    it = lax.broadcasted_iota(jnp.int32, d2.shape, 1)
    M = jnp.zeros(d2.shape, jnp.float32)
    for _ in range(_K):
        m = jnp.min(d2, axis=1, keepdims=True)
        idx = jnp.min(jnp.where(d2 == m, it, N), axis=1, keepdims=True)
        sel = it == idx
        M = M + sel.astype(jnp.float32)
        d2 = jnp.where(sel, jnp.inf, d2)
    return M


def _mlp(xr, S, Wi_ref, bi_ref, Wn_ref, bn_ref, Wf_ref, bf_ref):
    F0 = jnp.dot(xr, Wi_ref[...], preferred_element_type=jnp.float32) + bi_ref[...]
    G = F0 + jnp.dot(S - 6.0 * xr, Wn_ref[...],
                     preferred_element_type=jnp.float32) + 6.0 * bn_ref[...]
    return jnp.dot(G, Wf_ref[...], preferred_element_type=jnp.float32) + 7.0 * bf_ref[...]


def _tc_main_body(xq3_ref, xy_ref, x0_ref, depot_ref,
                  Wi_ref, bi_ref, Wn_ref, bn_ref, Wf_ref, bf_ref,
                  Wd_ref, bd_ref,
                  F_ref, dep_ref, sm_ref):
    j = pl.program_id(0)
    N = xy_ref.shape[1]
    nb_per_batch = N // _R
    xr = xq3_ref[...]
    xi = xr[:, 0:2]
    xj = xy_ref[0]
    dx = xi[:, 0:1] - xj[:, 0][None, :]
    dy = xi[:, 1:2] - xj[:, 1][None, :]
    M = _topk_mask(dx * dx + dy * dy, N)
    S = jnp.dot(M, x0_ref[...], preferred_element_type=jnp.float32)
    F = _mlp(xr, S, Wi_ref, bi_ref, Wn_ref, bn_ref, Wf_ref, bf_ref)
    F_ref[...] = F
    rowsum = jnp.sum(F, axis=0, keepdims=True)
    bmask = lax.broadcasted_iota(jnp.int32, (2, 1), 0) == j // nb_per_batch
    contrib = jnp.where(bmask, rowsum, 0.0)

    @pl.when(j == 0)
    def _():
        dep_ref[...] = jnp.dot(depot_ref[...], Wd_ref[...],
                               preferred_element_type=jnp.float32) + bd_ref[...]
        sm_ref[...] = contrib

    @pl.when(j != 0)
    def _():
        sm_ref[...] += contrib


def _tc_tail_body(xq3_ref, S_ref,
                  Wi_ref, bi_ref, Wn_ref, bn_ref, Wf_ref, bf_ref,
                  F_ref, sm_ref):
    j = pl.program_id(0)
    F = _mlp(xq3_ref[...], S_ref[...],
             Wi_ref, bi_ref, Wn_ref, bn_ref, Wf_ref, bf_ref)
    F_ref[...] = F
    rowsum = jnp.sum(F, axis=0, keepdims=True)

    @pl.when(j == 0)
    def _():
        sm_ref[...] = rowsum

    @pl.when(j != 0)
    def _():
        sm_ref[...] += rowsum


def _sc_select_kernel(n_rows):
    rows_per_tile = n_rows // _SUBCORES
    groups = rows_per_tile // _LANES
    mesh = plsc.VectorSubcoreMesh(core_axis_name="c", subcore_axis_name="s")

    def body(qx_hbm, qy_hbm, cx_hbm, cy_hbm, p0_hbm, p1_hbm, p2_hbm,
             s0_hbm, s1_hbm, s2_hbm,
             qx_v, qy_v, cx_v, cy_v, p0_v, p1_v, p2_v, s0_v, s1_v, s2_v):
        N = cx_hbm.shape[0]
        w = lax.axis_index("s") * 2 + lax.axis_index("c")
        base = w * rows_per_tile
        pltpu.sync_copy(qx_hbm.at[pl.ds(base, rows_per_tile)], qx_v)
        pltpu.sync_copy(qy_hbm.at[pl.ds(base, rows_per_tile)], qy_v)
        pltpu.sync_copy(cx_hbm, cx_v)
        pltpu.sync_copy(cy_hbm, cy_v)
        pltpu.sync_copy(p0_hbm, p0_v)
        pltpu.sync_copy(p1_hbm, p1_v)
        pltpu.sync_copy(p2_hbm, p2_v)
        for g in range(groups):
            xi = qx_v[pl.ds(g * _LANES, _LANES)]
            yi = qy_v[pl.ds(g * _LANES, _LANES)]
            inf = jnp.full((_LANES,), jnp.inf, jnp.float32)
            zero = jnp.zeros((_LANES,), jnp.int32)
            carry = (inf, inf, inf, inf, inf, inf,
                     zero, zero, zero, zero, zero, zero)

            def step(c, cr):
                b0, b1, b2, b3, b4, b5, i0, i1, i2, i3, i4, i5 = cr
                jv = jnp.full((_LANES,), c, jnp.int32)
                bx = plsc.load_gather(cx_v, [jv])
                by = plsc.load_gather(cy_v, [jv])
                dx = bx - xi
                dy = by - yi
                d2 = dx * dx + dy * dy
                c0 = d2 < b0
                c1 = d2 < b1
                c2 = d2 < b2
                c3 = d2 < b3
                c4 = d2 < b4
                c5 = d2 < b5
                nb5 = jnp.where(c5, jnp.where(c4, b4, d2), b5)
                ni5 = jnp.where(c5, jnp.where(c4, i4, jv), i5)
                nb4 = jnp.where(c4, jnp.where(c3, b3, d2), b4)
                ni4 = jnp.where(c4, jnp.where(c3, i3, jv), i4)
                nb3 = jnp.where(c3, jnp.where(c2, b2, d2), b3)
                ni3 = jnp.where(c3, jnp.where(c2, i2, jv), i3)
                nb2 = jnp.where(c2, jnp.where(c1, b1, d2), b2)
                ni2 = jnp.where(c2, jnp.where(c1, i1, jv), i2)
                nb1 = jnp.where(c1, jnp.where(c0, b0, d2), b1)
                ni1 = jnp.where(c1, jnp.where(c0, i0, jv), i1)
                nb0 = jnp.where(c0, d2, b0)
                ni0 = jnp.where(c0, jv, i0)
                return (nb0, nb1, nb2, nb3, nb4, nb5,
                        ni0, ni1, ni2, ni3, ni4, ni5)

            carry = lax.fori_loop(0, N, step, carry, unroll=4)
            idxs = carry[6:]
            s0 = jnp.zeros((_LANES,), jnp.float32)
            s1 = jnp.zeros((_LANES,), jnp.float32)
            s2 = jnp.zeros((_LANES,), jnp.float32)
            for ik in idxs:
                s0 = s0 + plsc.load_gather(p0_v, [ik])
                s1 = s1 + plsc.load_gather(p1_v, [ik])
                s2 = s2 + plsc.load_gather(p2_v, [ik])
            s0_v[pl.ds(g * _LANES, _LANES)] = s0
            s1_v[pl.ds(g * _LANES, _LANES)] = s1
            s2_v[pl.ds(g * _LANES, _LANES)] = s2
        pltpu.sync_copy(s0_v, s0_hbm.at[pl.ds(base, rows_per_tile)])
        pltpu.sync_copy(s1_v, s1_hbm.at[pl.ds(base, rows_per_tile)])
        pltpu.sync_copy(s2_v, s2_hbm.at[pl.ds(base, rows_per_tile)])

    out = jax.ShapeDtypeStruct((n_rows,), jnp.float32)
    vec = lambda n: pltpu.VMEM((n,), jnp.float32)
    return pl.kernel(
        body,
        mesh=mesh,
        compiler_params=pltpu.CompilerParams(needs_layout_passes=False),
        out_type=[out, out, out],
        scratch_types=[
            vec(rows_per_tile), vec(rows_per_tile),
            vec(4096), vec(4096), vec(4096), vec(4096), vec(4096),
            vec(rows_per_tile), vec(rows_per_tile), vec(rows_per_tile),
        ],
    )


def kernel(loc, deadline, depot, Wi, bi, Wn, bn, Wf, bf, Wd, bd):
    B, N, _ = loc.shape
    E = Wi.shape[1]
    x = jnp.concatenate([loc, deadline[:, :, None]], axis=2)
    x0 = x[0]
    bi2, bn2, bf2, bd2 = (v.reshape(1, E) for v in (bi, bn, bf, bd))
    n_tc = B * N - _N_SC
    xq3 = x.reshape(B * N, 3)
    wspecs = [
        pl.BlockSpec((3, E), lambda j: (0, 0)),
        pl.BlockSpec((1, E), lambda j: (0, 0)),
        pl.BlockSpec((3, E), lambda j: (0, 0)),
        pl.BlockSpec((1, E), lambda j: (0, 0)),
        pl.BlockSpec((E, E), lambda j: (0, 0)),
        pl.BlockSpec((1, E), lambda j: (0, 0)),
    ]

    tail = N - _N_SC
    s0, s1, s2 = _sc_select_kernel(_N_SC)(
        loc[1, tail:, 0], loc[1, tail:, 1], loc[1, :, 0], loc[1, :, 1],
        x0[:, 0], x0[:, 1], x0[:, 2])
    S_sc = jnp.stack([s0, s1, s2], axis=-1)

    F_tc, dep, sm_tc = pl.pallas_call(
        _tc_main_body,
        grid=(n_tc // _R,),
        in_specs=[
            pl.BlockSpec((_R, 3), lambda j: (j, 0)),
            pl.BlockSpec((1, N, 2), lambda j: (j // (N // _R), 0, 0)),
            pl.BlockSpec((N, 3), lambda j: (0, 0)),
            pl.BlockSpec((2, 2), lambda j: (0, 0)),
        ] + wspecs + [
            pl.BlockSpec((2, E), lambda j: (0, 0)),
            pl.BlockSpec((1, E), lambda j: (0, 0)),
        ],
        out_specs=[
            pl.BlockSpec((_R, E), lambda j: (j, 0)),
            pl.BlockSpec((2, E), lambda j: (0, 0)),
            pl.BlockSpec((2, E), lambda j: (0, 0)),
        ],
        out_shape=[
            jax.ShapeDtypeStruct((n_tc, E), jnp.float32),
            jax.ShapeDtypeStruct((2, E), jnp.float32),
            jax.ShapeDtypeStruct((2, E), jnp.float32),
        ],
    )(xq3[:n_tc], loc, x0, depot, Wi, bi2, Wn, bn2, Wf, bf2, Wd, bd2)

    F_sc, sm_sc = pl.pallas_call(
        _tc_tail_body,
        grid=(_N_SC // _R,),
        in_specs=[
            pl.BlockSpec((_R, 3), lambda j: (j, 0)),
            pl.BlockSpec((_R, 3), lambda j: (j, 0)),
        ] + wspecs,
        out_specs=[
            pl.BlockSpec((_R, E), lambda j: (j, 0)),
            pl.BlockSpec((1, E), lambda j: (0, 0)),
        ],
        out_shape=[
            jax.ShapeDtypeStruct((_N_SC, E), jnp.float32),
            jax.ShapeDtypeStruct((1, E), jnp.float32),
        ],
    )(xq3[n_tc:], S_sc, Wi, bi2, Wn, bn2, Wf, bf2)

    F = jnp.concatenate([F_tc, F_sc], axis=0).reshape(B, N, E)
    h = jnp.concatenate([dep[:, None, :], F], axis=1)
    sm = sm_tc + jnp.concatenate(
        [jnp.zeros((1, E), jnp.float32), sm_sc], axis=0)
    mean = (dep + sm) / (N + 1)
    return (h, mean)


# speedup vs baseline: 2.6340x; 1.0499x over previous
import jax
import jax.numpy as jnp
from jax import lax
from jax.experimental import pallas as pl
from jax.experimental.pallas import tpu as pltpu
from jax.experimental.pallas import tpu_sc as plsc

_R = 256
_K = 6
_LANES = 16
_SUBCORES = 32
_N_SC = 3584


def _topk_mask(d2, N, self_idx):
    it = lax.broadcasted_iota(jnp.int32, d2.shape, 1)
    sel0 = it == self_idx
    M = sel0.astype(jnp.float32)
    d2 = jnp.where(sel0, jnp.inf, d2)
    for _ in range(_K - 1):
        m = jnp.min(d2, axis=1, keepdims=True)
        idx = jnp.min(jnp.where(d2 == m, it, N), axis=1, keepdims=True)
        sel = it == idx
        M = M + sel.astype(jnp.float32)
        d2 = jnp.where(sel, jnp.inf, d2)
    return M


def _mlp(xr, S, Wi_ref, bi_ref, Wn_ref, bn_ref, Wf_ref, bf_ref):
    F0 = jnp.dot(xr, Wi_ref[...], preferred_element_type=jnp.float32) + bi_ref[...]
    G = F0 + jnp.dot(S - 6.0 * xr, Wn_ref[...],
                     preferred_element_type=jnp.float32) + 6.0 * bn_ref[...]
    return jnp.dot(G, Wf_ref[...], preferred_element_type=jnp.float32) + 7.0 * bf_ref[...]


def _tc_main_body(xq3_ref, xy_ref, x0_ref, depot_ref,
                  Wi_ref, bi_ref, Wn_ref, bn_ref, Wf_ref, bf_ref,
                  Wd_ref, bd_ref,
                  F_ref, dep_ref, sm_ref):
    j = pl.program_id(0)
    N = xy_ref.shape[1]
    nb_per_batch = N // _R
    xr = xq3_ref[...]
    xi = xr[:, 0:2]
    xj = xy_ref[0]
    dx = xi[:, 0:1] - xj[:, 0][None, :]
    dy = xi[:, 1:2] - xj[:, 1][None, :]
    self_idx = ((j % nb_per_batch) * _R
                + lax.broadcasted_iota(jnp.int32, (_R, 1), 0))
    M = _topk_mask(dx * dx + dy * dy, N, self_idx)
    S = jnp.dot(M, x0_ref[...], preferred_element_type=jnp.float32)
    F = _mlp(xr, S, Wi_ref, bi_ref, Wn_ref, bn_ref, Wf_ref, bf_ref)
    F_ref[...] = F
    rowsum = jnp.sum(F, axis=0, keepdims=True)
    bmask = lax.broadcasted_iota(jnp.int32, (2, 1), 0) == j // nb_per_batch
    contrib = jnp.where(bmask, rowsum, 0.0)

    @pl.when(j == 0)
    def _():
        dep_ref[...] = jnp.dot(depot_ref[...], Wd_ref[...],
                               preferred_element_type=jnp.float32) + bd_ref[...]
        sm_ref[...] = contrib

    @pl.when(j != 0)
    def _():
        sm_ref[...] += contrib


def _tc_tail_body(xq3_ref, S_ref,
                  Wi_ref, bi_ref, Wn_ref, bn_ref, Wf_ref, bf_ref,
                  F_ref, sm_ref):
    j = pl.program_id(0)
    F = _mlp(xq3_ref[...], S_ref[...],
             Wi_ref, bi_ref, Wn_ref, bn_ref, Wf_ref, bf_ref)
    F_ref[...] = F
    rowsum = jnp.sum(F, axis=0, keepdims=True)

    @pl.when(j == 0)
    def _():
        sm_ref[...] = rowsum

    @pl.when(j != 0)
    def _():
        sm_ref[...] += rowsum


def _sc_select_kernel(n_rows, self_off):
    rows_per_tile = n_rows // _SUBCORES
    groups = rows_per_tile // _LANES
    mesh = plsc.VectorSubcoreMesh(core_axis_name="c", subcore_axis_name="s")

    def body(qx_hbm, qy_hbm, cx_hbm, cy_hbm, p0_hbm, p1_hbm, p2_hbm,
             s0_hbm, s1_hbm, s2_hbm,
             qx_v, qy_v, cx_v, cy_v, p0_v, p1_v, p2_v, s0_v, s1_v, s2_v):
        N = cx_hbm.shape[0]
        w = lax.axis_index("s") * 2 + lax.axis_index("c")
        base = w * rows_per_tile
        pltpu.sync_copy(qx_hbm.at[pl.ds(base, rows_per_tile)], qx_v)
        pltpu.sync_copy(qy_hbm.at[pl.ds(base, rows_per_tile)], qy_v)
        pltpu.sync_copy(cx_hbm, cx_v)
        pltpu.sync_copy(cy_hbm, cy_v)
        pltpu.sync_copy(p0_hbm, p0_v)
        pltpu.sync_copy(p1_hbm, p1_v)
        pltpu.sync_copy(p2_hbm, p2_v)
        for g in range(groups):
            xi = qx_v[pl.ds(g * _LANES, _LANES)]
            yi = qy_v[pl.ds(g * _LANES, _LANES)]
            self_i = (jnp.full((_LANES,), self_off + base + g * _LANES,
                               jnp.int32) + lax.iota(jnp.int32, 16))
            inf = jnp.full((_LANES,), jnp.inf, jnp.float32)
            zero = jnp.zeros((_LANES,), jnp.int32)
            carry = (inf, inf, inf, inf, inf,
                     zero, zero, zero, zero, zero)

            def step(c, cr):
                b1, b2, b3, b4, b5, i1, i2, i3, i4, i5 = cr
                jv = jnp.full((_LANES,), c, jnp.int32)
                bx = plsc.load_gather(cx_v, [jv])
                by = plsc.load_gather(cy_v, [jv])
                dx = bx - xi
                dy = by - yi
                d2 = dx * dx + dy * dy
                d2 = jnp.where(jv == self_i, jnp.inf, d2)
                c1 = d2 < b1
                c2 = d2 < b2
                c3 = d2 < b3
                c4 = d2 < b4
                c5 = d2 < b5
                nb5 = jnp.where(c5, jnp.where(c4, b4, d2), b5)
                ni5 = jnp.where(c5, jnp.where(c4, i4, jv), i5)
                nb4 = jnp.where(c4, jnp.where(c3, b3, d2), b4)
                ni4 = jnp.where(c4, jnp.where(c3, i3, jv), i4)
                nb3 = jnp.where(c3, jnp.where(c2, b2, d2), b3)
                ni3 = jnp.where(c3, jnp.where(c2, i2, jv), i3)
                nb2 = jnp.where(c2, jnp.where(c1, b1, d2), b2)
                ni2 = jnp.where(c2, jnp.where(c1, i1, jv), i2)
                nb1 = jnp.where(c1, d2, b1)
                ni1 = jnp.where(c1, jv, i1)
                return (nb1, nb2, nb3, nb4, nb5,
                        ni1, ni2, ni3, ni4, ni5)

            carry = lax.fori_loop(0, N, step, carry, unroll=4)
            idxs = (self_i,) + carry[5:]
            s0 = jnp.zeros((_LANES,), jnp.float32)
            s1 = jnp.zeros((_LANES,), jnp.float32)
            s2 = jnp.zeros((_LANES,), jnp.float32)
            for ik in idxs:
                s0 = s0 + plsc.load_gather(p0_v, [ik])
                s1 = s1 + plsc.load_gather(p1_v, [ik])
                s2 = s2 + plsc.load_gather(p2_v, [ik])
            s0_v[pl.ds(g * _LANES, _LANES)] = s0
            s1_v[pl.ds(g * _LANES, _LANES)] = s1
            s2_v[pl.ds(g * _LANES, _LANES)] = s2
        pltpu.sync_copy(s0_v, s0_hbm.at[pl.ds(base, rows_per_tile)])
        pltpu.sync_copy(s1_v, s1_hbm.at[pl.ds(base, rows_per_tile)])
        pltpu.sync_copy(s2_v, s2_hbm.at[pl.ds(base, rows_per_tile)])

    out = jax.ShapeDtypeStruct((n_rows,), jnp.float32)
    vec = lambda n: pltpu.VMEM((n,), jnp.float32)
    return pl.kernel(
        body,
        mesh=mesh,
        compiler_params=pltpu.CompilerParams(needs_layout_passes=False),
        out_type=[out, out, out],
        scratch_types=[
            vec(rows_per_tile), vec(rows_per_tile),
            vec(4096), vec(4096), vec(4096), vec(4096), vec(4096),
            vec(rows_per_tile), vec(rows_per_tile), vec(rows_per_tile),
        ],
    )


def kernel(loc, deadline, depot, Wi, bi, Wn, bn, Wf, bf, Wd, bd):
    B, N, _ = loc.shape
    E = Wi.shape[1]
    x = jnp.concatenate([loc, deadline[:, :, None]], axis=2)
    x0 = x[0]
    bi2, bn2, bf2, bd2 = (v.reshape(1, E) for v in (bi, bn, bf, bd))
    n_tc = B * N - _N_SC
    xq3 = x.reshape(B * N, 3)
    wspecs = [
        pl.BlockSpec((3, E), lambda j: (0, 0)),
        pl.BlockSpec((1, E), lambda j: (0, 0)),
        pl.BlockSpec((3, E), lambda j: (0, 0)),
        pl.BlockSpec((1, E), lambda j: (0, 0)),
        pl.BlockSpec((E, E), lambda j: (0, 0)),
        pl.BlockSpec((1, E), lambda j: (0, 0)),
    ]

    tail = N - _N_SC
    s0, s1, s2 = _sc_select_kernel(_N_SC, tail)(
        loc[1, tail:, 0], loc[1, tail:, 1], loc[1, :, 0], loc[1, :, 1],
        x0[:, 0], x0[:, 1], x0[:, 2])
    S_sc = jnp.stack([s0, s1, s2], axis=-1)

    F_tc, dep, sm_tc = pl.pallas_call(
        _tc_main_body,
        grid=(n_tc // _R,),
        in_specs=[
            pl.BlockSpec((_R, 3), lambda j: (j, 0)),
            pl.BlockSpec((1, N, 2), lambda j: (j // (N // _R), 0, 0)),
            pl.BlockSpec((N, 3), lambda j: (0, 0)),
            pl.BlockSpec((2, 2), lambda j: (0, 0)),
        ] + wspecs + [
            pl.BlockSpec((2, E), lambda j: (0, 0)),
            pl.BlockSpec((1, E), lambda j: (0, 0)),
        ],
        out_specs=[
            pl.BlockSpec((_R, E), lambda j: (j, 0)),
            pl.BlockSpec((2, E), lambda j: (0, 0)),
            pl.BlockSpec((2, E), lambda j: (0, 0)),
        ],
        out_shape=[
            jax.ShapeDtypeStruct((n_tc, E), jnp.float32),
            jax.ShapeDtypeStruct((2, E), jnp.float32),
            jax.ShapeDtypeStruct((2, E), jnp.float32),
        ],
    )(xq3[:n_tc], loc, x0, depot, Wi, bi2, Wn, bn2, Wf, bf2, Wd, bd2)

    F_sc, sm_sc = pl.pallas_call(
        _tc_tail_body,
        grid=(_N_SC // _R,),
        in_specs=[
            pl.BlockSpec((_R, 3), lambda j: (j, 0)),
            pl.BlockSpec((_R, 3), lambda j: (j, 0)),
        ] + wspecs,
        out_specs=[
            pl.BlockSpec((_R, E), lambda j: (j, 0)),
            pl.BlockSpec((1, E), lambda j: (0, 0)),
        ],
        out_shape=[
            jax.ShapeDtypeStruct((_N_SC, E), jnp.float32),
            jax.ShapeDtypeStruct((1, E), jnp.float32),
        ],
    )(xq3[n_tc:], S_sc, Wi, bi2, Wn, bn2, Wf, bf2)

    F = jnp.concatenate([F_tc, F_sc], axis=0).reshape(B, N, E)
    h = jnp.concatenate([dep[:, None, :], F], axis=1)
    sm = sm_tc + jnp.concatenate(
        [jnp.zeros((1, E), jnp.float32), sm_sc], axis=0)
    mean = (dep + sm) / (N + 1)
    return (h, mean)
